# baseline (speedup 1.0000x reference)
import functools

import jax
import jax.numpy as jnp
from jax import lax
from jax.experimental import pallas as pl
from jax.experimental.pallas import tpu as pltpu
from jax.experimental.pallas import tpu_sc as plsc

NC = 2
NS = 16
NW = NC * NS
L = 16
G = 8
ROWS = 128
COLS = 100000
R = 200
NCH = COLS // R
CPW = 16
DEPTH = 4

_mesh = plsc.VectorSubcoreMesh(core_axis_name="c", subcore_axis_name="s")


@functools.partial(
    pl.kernel,
    out_type=(
        jax.ShapeDtypeStruct((NW, G * L), jnp.float32),
        jax.ShapeDtypeStruct((NW, G * L), jnp.int32),
    ),
    mesh=_mesh,
    compiler_params=pltpu.CompilerParams(use_tc_tiling_on_sc=True),
    scratch_types=[
        pltpu.VMEM((R, ROWS), jnp.float32),
        pltpu.VMEM((R, ROWS), jnp.float32),
        pltpu.VMEM((R, ROWS), jnp.float32),
        pltpu.VMEM((R, ROWS), jnp.float32),
        pltpu.VMEM((G * L,), jnp.float32),
        pltpu.VMEM((G * L,), jnp.int32),
        pltpu.SemaphoreType.DMA,
        pltpu.SemaphoreType.DMA,
        pltpu.SemaphoreType.DMA,
        pltpu.SemaphoreType.DMA,
    ],
)
def _argmax_sc(xt_hbm, outv_hbm, outi_hbm, buf0, buf1, buf2, buf3, resv, resi,
               sem0, sem1, sem2, sem3):
    wid = lax.axis_index("s") * NC + lax.axis_index("c")

    def chunk_r0(i):
        return jnp.minimum(wid + NW * i, NCH - 1) * R

    def chunk_src(i):
        return xt_hbm.at[pl.ds(chunk_r0(i), R), :]

    bufs = (buf0, buf1, buf2, buf3)
    sems = (sem0, sem1, sem2, sem3)

    for i in range(DEPTH - 1):
        pltpu.async_copy(chunk_src(i), bufs[i], sems[i])

    accv = tuple(jnp.full((L,), -jnp.inf, jnp.float32) for _ in range(G))
    acci = tuple(jnp.zeros((L,), jnp.int32) for _ in range(G))

    for i in range(CPW):
        s = i % DEPTH
        if i + DEPTH - 1 < CPW:
            j = i + DEPTH - 1
            pltpu.async_copy(chunk_src(j), bufs[j % DEPTH], sems[j % DEPTH])
        pltpu.make_async_copy(chunk_src(i), bufs[s], sems[s]).wait()

        def body(k, carry, buf=bufs[s]):
            accv, acci, cur = carry
            nv, ni = [], []
            for g in range(G):
                v = buf[k, pl.ds(g * L, L)]
                pred = v > accv[g]
                nv.append(jnp.where(pred, v, accv[g]))
                ni.append(jnp.where(pred, cur, acci[g]))
            return tuple(nv), tuple(ni), cur + 1

        cur0 = jnp.full((L,), chunk_r0(i), jnp.int32)
        accv, acci, _ = lax.fori_loop(
            0, R, body, (accv, acci, cur0), unroll=4
        )

    for g in range(G):
        resv[pl.ds(g * L, L)] = accv[g]
        resi[pl.ds(g * L, L)] = acci[g]
    pltpu.sync_copy(resv, outv_hbm.at[wid])
    pltpu.sync_copy(resi, outi_hbm.at[wid])


def kernel(m_logits):
    outv, outi = _argmax_sc(m_logits.T)
    bv, bi = outv[0], outi[0]
    for w in range(1, NW):
        pred = (outv[w] > bv) | ((outv[w] == bv) & (outi[w] < bi))
        bv = jnp.where(pred, outv[w], bv)
        bi = jnp.where(pred, outi[w], bi)
    return bi.reshape(ROWS, 1)


# speedup vs baseline: 1.2691x; 1.2691x over previous
import functools

import jax
import jax.numpy as jnp
from jax import lax
from jax.experimental import pallas as pl
from jax.experimental.pallas import tpu as pltpu
from jax.experimental.pallas import tpu_sc as plsc

NC = 2
NS = 16
NW = NC * NS
L = 16
G = 8
ROWS = 128
COLS = 100000
R = 400
NCH = COLS // R
CPW = 8

_mesh = plsc.VectorSubcoreMesh(core_axis_name="c", subcore_axis_name="s")


@functools.partial(
    pl.kernel,
    out_type=(
        jax.ShapeDtypeStruct((NW, G * L), jnp.float32),
        jax.ShapeDtypeStruct((NW, G * L), jnp.int32),
    ),
    mesh=_mesh,
    compiler_params=pltpu.CompilerParams(use_tc_tiling_on_sc=True),
    scratch_types=[
        pltpu.VMEM((R, ROWS), jnp.float32),
        pltpu.VMEM((R, ROWS), jnp.float32),
        pltpu.VMEM((G * L,), jnp.float32),
        pltpu.VMEM((G * L,), jnp.int32),
        pltpu.SemaphoreType.DMA,
        pltpu.SemaphoreType.DMA,
    ],
)
def _argmax_sc(xt_hbm, outv_hbm, outi_hbm, buf0, buf1, resv, resi, sem0, sem1):
    wid = lax.axis_index("s") * NC + lax.axis_index("c")

    def chunk_r0(i):
        return jnp.minimum(wid + NW * i, NCH - 1) * R

    def chunk_src(i):
        return xt_hbm.at[pl.ds(chunk_r0(i), R), :]

    bufs = (buf0, buf1)
    sems = (sem0, sem1)

    pltpu.async_copy(chunk_src(0), buf0, sem0)

    accv = tuple(jnp.full((L,), -jnp.inf, jnp.float32) for _ in range(G))
    acci = tuple(jnp.zeros((L,), jnp.int32) for _ in range(G))

    for i in range(CPW):
        s = i & 1
        if i + 1 < CPW:
            pltpu.async_copy(chunk_src(i + 1), bufs[1 - s], sems[1 - s])
        pltpu.make_async_copy(chunk_src(i), bufs[s], sems[s]).wait()

        def body(k, carry, buf=bufs[s]):
            accv, acci, cur = carry
            nv, ni = [], []
            for g in range(G):
                v = buf[k, pl.ds(g * L, L)]
                pred = v > accv[g]
                nv.append(jnp.where(pred, v, accv[g]))
                ni.append(jnp.where(pred, cur, acci[g]))
            return tuple(nv), tuple(ni), cur + 1

        cur0 = jnp.full((L,), chunk_r0(i), jnp.int32)
        accv, acci, _ = lax.fori_loop(
            0, R, body, (accv, acci, cur0), unroll=2
        )

    for g in range(G):
        resv[pl.ds(g * L, L)] = accv[g]
        resi[pl.ds(g * L, L)] = acci[g]
    pltpu.sync_copy(resv, outv_hbm.at[wid])
    pltpu.sync_copy(resi, outi_hbm.at[wid])


def kernel(m_logits):
    outv, outi = _argmax_sc(m_logits.T)
    bv, bi = outv[0], outi[0]
    for w in range(1, NW):
        pred = (outv[w] > bv) | ((outv[w] == bv) & (outi[w] < bi))
        bv = jnp.where(pred, outv[w], bv)
        bi = jnp.where(pred, outi[w], bi)
    return bi.reshape(ROWS, 1)
